# baseline (speedup 1.0000x reference)
import functools

import jax
import jax.numpy as jnp
from jax import lax
from jax.experimental import pallas as pl
from jax.experimental.pallas import tpu as pltpu
from jax.experimental.pallas import tpu_sc as plsc

B, L, H = 4096, 50, 768
N = B * L
NF = 1000
NC, NS = 2, 16
NW = NC * NS
NPW = N // NW
K = 16
CH = NPW // K
D = 4
VPR = H // 16
SR = 8
SCH = NF // SR

_mesh = plsc.VectorSubcoreMesh(core_axis_name="c", subcore_axis_name="s")


@functools.partial(
    pl.kernel,
    mesh=_mesh,
    out_type=jax.ShapeDtypeStruct((N, H), jnp.float32),
    scratch_types=[
        pltpu.VMEM((NPW,), jnp.int32),
        pltpu.VMEM((NPW,), jnp.int32),
        [pltpu.VMEM((K, H), jnp.float32) for _ in range(D)],
        [pltpu.VMEM((K, H), jnp.float32) for _ in range(D)],
        [pltpu.SemaphoreType.DMA for _ in range(D)],
        [pltpu.SemaphoreType.DMA for _ in range(D)],
        [pltpu.SemaphoreType.DMA for _ in range(D)],
    ],
)
def _embed(ids_hbm, flg_hbm, itab_hbm, ftab_hbm, out_hbm,
           ids_v, flg_v, irows, frows, sem_i, sem_f, sem_o):
    sid = lax.axis_index("s")
    wid = sid * NC + lax.axis_index("c")
    base = wid * NPW

    pltpu.sync_copy(ids_hbm.at[pl.ds(base, NPW)], ids_v)
    pltpu.sync_copy(flg_hbm.at[pl.ds(base, NPW)], flg_v)

    def fire(c, b):
        off = c * K
        pltpu.async_copy(itab_hbm.at[ids_v.at[pl.ds(off, K)]], irows[b], sem_i[b])
        pltpu.async_copy(ftab_hbm.at[flg_v.at[pl.ds(off, K)]], frows[b], sem_f[b])

    def drain_gather(b):
        pltpu.make_async_copy(itab_hbm.at[ids_v.at[pl.ds(0, K)]], irows[b], sem_i[b]).wait()
        pltpu.make_async_copy(ftab_hbm.at[flg_v.at[pl.ds(0, K)]], frows[b], sem_f[b]).wait()

    def drain_out(b):
        pltpu.make_async_copy(irows[b], out_hbm.at[pl.ds(base, K)], sem_o[b]).wait()

    fire(0, 0)
    fire(1, 1)

    def outer(g, carry):
        for b in range(D):
            c = g * D + b
            b2 = (b + 2) % D

            @pl.when(c + 2 < CH)
            def _():
                @pl.when(c >= 2)
                def _():
                    drain_out(b2)
                fire(c + 2, b2)

            drain_gather(b)

            def row(r, rc):
                for v in range(VPR):
                    sl = pl.ds(v * 16, 16)
                    irows[b][r, sl] = irows[b][r, sl] + frows[b][r, sl]
                return rc

            lax.fori_loop(0, K, row, 0)
            pltpu.async_copy(irows[b], out_hbm.at[pl.ds(base + c * K, K)], sem_o[b])
        return carry

    lax.fori_loop(0, CH // D, outer, 0)

    drain_out((CH - 2) % D)
    drain_out((CH - 1) % D)


def kernel(item_ids, flags, item_table, flag_table):
    ids = item_ids.reshape(N).astype(jnp.int32)
    flg = flags.reshape(N).astype(jnp.int32)
    out = _embed(ids, flg, item_table, flag_table)
    return out.reshape(B, L, H)


# speedup vs baseline: 1.0006x; 1.0006x over previous
import functools

import jax
import jax.numpy as jnp
from jax import lax
from jax.experimental import pallas as pl
from jax.experimental.pallas import tpu as pltpu
from jax.experimental.pallas import tpu_sc as plsc

B, L, H = 4096, 50, 768
N = B * L
NC, NS = 2, 16
NW = NC * NS
NPW = N // NW
K = 8
CH = NPW // K
D = 8
LA = 4
VPR = H // 16

assert CH % D == 0 and LA < D

_mesh = plsc.VectorSubcoreMesh(core_axis_name="c", subcore_axis_name="s")


@functools.partial(
    pl.kernel,
    mesh=_mesh,
    out_type=jax.ShapeDtypeStruct((N, H), jnp.float32),
    scratch_types=[
        pltpu.VMEM((NPW,), jnp.int32),
        pltpu.VMEM((NPW,), jnp.int32),
        [pltpu.VMEM((K, H), jnp.float32) for _ in range(D)],
        [pltpu.VMEM((K, H), jnp.float32) for _ in range(D)],
        [pltpu.SemaphoreType.DMA for _ in range(D)],
        [pltpu.SemaphoreType.DMA for _ in range(D)],
        [pltpu.SemaphoreType.DMA for _ in range(D)],
    ],
)
def _embed(ids_hbm, flg_hbm, itab_hbm, ftab_hbm, out_hbm,
           ids_v, flg_v, irows, frows, sem_i, sem_f, sem_o):
    wid = lax.axis_index("s") * NC + lax.axis_index("c")
    base = wid * NPW
    pltpu.sync_copy(ids_hbm.at[pl.ds(base, NPW)], ids_v)
    pltpu.sync_copy(flg_hbm.at[pl.ds(base, NPW)], flg_v)

    def fire(c, b):
        off = c * K
        pltpu.async_copy(itab_hbm.at[ids_v.at[pl.ds(off, K)]], irows[b], sem_i[b])
        pltpu.async_copy(ftab_hbm.at[flg_v.at[pl.ds(off, K)]], frows[b], sem_f[b])

    def drain_gather(b):
        pltpu.make_async_copy(itab_hbm.at[ids_v.at[pl.ds(0, K)]], irows[b], sem_i[b]).wait()
        pltpu.make_async_copy(ftab_hbm.at[flg_v.at[pl.ds(0, K)]], frows[b], sem_f[b]).wait()

    def drain_out(b):
        pltpu.make_async_copy(irows[b], out_hbm.at[pl.ds(base, K)], sem_o[b]).wait()

    for c0 in range(LA):
        fire(c0, c0)

    def outer(g, carry):
        for b in range(D):
            c = g * D + b
            b2 = (b + LA) % D

            @pl.when(c + LA < CH)
            def _():
                @pl.when(c >= D - LA)
                def _():
                    drain_out(b2)
                fire(c + LA, b2)

            drain_gather(b)

            def row(r, rc):
                for v in range(VPR):
                    sl = pl.ds(v * 16, 16)
                    irows[b][r, sl] = irows[b][r, sl] + frows[b][r, sl]
                return rc

            lax.fori_loop(0, K, row, 0)
            pltpu.async_copy(irows[b], out_hbm.at[pl.ds(base + c * K, K)], sem_o[b])
        return carry

    lax.fori_loop(0, CH // D, outer, 0)

    for c0 in range(CH + LA - D, CH):
        drain_out(c0 % D)


def kernel(item_ids, flags, item_table, flag_table):
    ids = item_ids.reshape(N).astype(jnp.int32)
    flg = flags.reshape(N).astype(jnp.int32)
    out = _embed(ids, flg, item_table, flag_table)
    return out.reshape(B, L, H)
